# Initial kernel scaffold; baseline (speedup 1.0000x reference)
import jax
import jax.numpy as jnp
from jax.experimental import pallas as pl


def kernel(x, edge_index, batch, W1, b1, W2, b2, Wa, ba, Wl1, bl1, Wl2, bl2, Wc, bc):
    raise NotImplementedError("write your pallas kernel here")


# speedup vs baseline: 19.5474x; 19.5474x over previous
import functools

import jax
import jax.numpy as jnp
from jax import lax
from jax.experimental import pallas as pl
from jax.experimental.pallas import tpu as pltpu
from jax.experimental.pallas import tpu_sc as plsc

N = 10000
E = 320000
F = 128
G = 16
C = 2

NC = 2
NS = 16
NW = NC * NS
EPT = E // NW
K = 80
NCH = EPT // K
NPAD = 10240
RPT = NPAD // NS
ZR = 128
DW = 16

_vec_mesh = plsc.VectorSubcoreMesh(core_axis_name="c", subcore_axis_name="s")


@functools.partial(
    pl.kernel,
    out_type=jax.ShapeDtypeStruct((NC, NPAD, DW), jnp.float32),
    mesh=_vec_mesh,
    scratch_types=[
        pltpu.VMEM((NCH, K), jnp.int32),
        pltpu.VMEM((K, DW), jnp.float32),
        pltpu.VMEM((RPT, DW), jnp.float32),
        pltpu.VMEM_SHARED((NPAD, DW), jnp.float32),
    ],
    compiler_params=pltpu.CompilerParams(use_tc_tiling_on_sc=False),
)
def _sc_degree(dst_hbm, out_hbm, dst_v, ones_v, zero_v, acc):
    cid = lax.axis_index("c")
    sid = lax.axis_index("s")
    wid = cid * NS + sid

    pltpu.sync_copy(dst_hbm.at[wid], dst_v)

    @pl.loop(0, K)
    def _(r):
        ones_v[r, :] = jnp.ones((DW,), jnp.float32)

    @pl.loop(0, RPT)
    def _(r):
        zero_v[r, :] = jnp.zeros((DW,), jnp.float32)

    pltpu.sync_copy(zero_v, acc.at[pl.ds(sid * RPT, RPT)])
    plsc.subcore_barrier()

    @pl.loop(0, NCH)
    def _(j):
        pltpu.sync_copy(ones_v, acc.at[dst_v.at[j]], add=True)

    plsc.subcore_barrier()
    pltpu.sync_copy(acc.at[pl.ds(sid * RPT, RPT)], zero_v)
    pltpu.sync_copy(zero_v, out_hbm.at[cid, pl.ds(sid * RPT, RPT)])


@functools.partial(
    pl.kernel,
    out_type=jax.ShapeDtypeStruct((NC, NPAD, F), jnp.float32),
    mesh=_vec_mesh,
    scratch_types=[
        pltpu.VMEM((NCH, K), jnp.int32),
        pltpu.VMEM((NCH, K), jnp.int32),
        pltpu.VMEM((K, F), jnp.float32),
        pltpu.VMEM_SHARED((NPAD, F), jnp.float32),
        pltpu.SemaphoreType.DMA,
    ],
    compiler_params=pltpu.CompilerParams(use_tc_tiling_on_sc=False),
)
def _sc_edge_scatter(table_hbm, src_hbm, dst_hbm, out_hbm,
                     src_v, dst_v, rows_v, acc, sem):
    cid = lax.axis_index("c")
    sid = lax.axis_index("s")
    wid = cid * NS + sid

    pltpu.sync_copy(src_hbm.at[wid], src_v)
    pltpu.sync_copy(dst_hbm.at[wid], dst_v)

    @pl.loop(0, K)
    def _(r):
        @pl.loop(0, F // 16)
        def _(j):
            rows_v[r, pl.ds(j * 16, 16)] = jnp.zeros((16,), jnp.float32)

    @pl.loop(0, RPT // K)
    def _(b):
        pltpu.sync_copy(rows_v, acc.at[pl.ds(sid * RPT + b * K, K)])

    plsc.subcore_barrier()

    @pl.loop(0, NCH)
    def _(j):
        pltpu.async_copy(table_hbm.at[src_v.at[j]], rows_v, sem).wait()
        pltpu.sync_copy(rows_v, acc.at[dst_v.at[j]], add=True)

    plsc.subcore_barrier()

    @pl.loop(0, RPT // K)
    def _(b):
        pltpu.sync_copy(acc.at[pl.ds(sid * RPT + b * K, K)], rows_v)
        pltpu.sync_copy(rows_v,
                        out_hbm.at[cid, pl.ds(sid * RPT + b * K, K)])


def _tc_prescale(deg2, x, W1):
    RB = 1000

    def body(deg_ref, x_ref, w_ref, t1d_ref, dinv_ref):
        deg = deg_ref[0, :, 0:1] + deg_ref[1, :, 0:1] + 1.0
        dinv = lax.rsqrt(jnp.maximum(deg, 1.0))
        t = jnp.dot(x_ref[...], w_ref[...], preferred_element_type=jnp.float32)
        t1d_ref[...] = t * dinv
        dinv_ref[...] = dinv

    return pl.pallas_call(
        body,
        grid=(N // RB,),
        in_specs=[
            pl.BlockSpec((NC, RB, DW), lambda i: (0, i, 0)),
            pl.BlockSpec((RB, F), lambda i: (i, 0)),
            pl.BlockSpec((F, F), lambda i: (0, 0)),
        ],
        out_specs=[
            pl.BlockSpec((RB, F), lambda i: (i, 0)),
            pl.BlockSpec((RB, 1), lambda i: (i, 0)),
        ],
        out_shape=[
            jax.ShapeDtypeStruct((N, F), jnp.float32),
            jax.ShapeDtypeStruct((N, 1), jnp.float32),
        ],
    )(deg2, x, W1)


def _tc_combine1(s1, t1d, dinv, b1_8, W2):
    RB = 1000

    def body(s_ref, t1d_ref, dinv_ref, b1_ref, w2_ref, t2d_ref):
        s = s_ref[0] + s_ref[1] + t1d_ref[...]
        h1 = jnp.maximum(s * dinv_ref[...] + b1_ref[0:1, :], 0.0)
        t2 = jnp.dot(h1, w2_ref[...], preferred_element_type=jnp.float32)
        t2d_ref[...] = t2 * dinv_ref[...]

    return pl.pallas_call(
        body,
        grid=(N // RB,),
        in_specs=[
            pl.BlockSpec((NC, RB, F), lambda i: (0, i, 0)),
            pl.BlockSpec((RB, F), lambda i: (i, 0)),
            pl.BlockSpec((RB, 1), lambda i: (i, 0)),
            pl.BlockSpec((8, F), lambda i: (0, 0)),
            pl.BlockSpec((F, F), lambda i: (0, 0)),
        ],
        out_specs=pl.BlockSpec((RB, F), lambda i: (i, 0)),
        out_shape=jax.ShapeDtypeStruct((N, F), jnp.float32),
    )(s1, t1d, dinv, b1_8, W2)


def _tc_head(s2, t2d, dinv, batch_col, b2_8, Wa8, ba_8, Wl1, bl1_8,
             Wl2, bl2_8, Wc128, bc_8):
    def body(s_ref, t2d_ref, dinv_ref, batch_ref, b2_ref, wa_ref, ba_ref,
             wl1_ref, bl1_ref, wl2_ref, bl2_ref, wc_ref, bc_ref, o_ref):
        h2 = ((s_ref[0] + s_ref[1] + t2d_ref[...]) * dinv_ref[...]
              + b2_ref[0:1, :])
        a8 = jnp.dot(h2, wa_ref[...], preferred_element_type=jnp.float32)
        a = a8[:, 0:1] + ba_ref[0:1, 0:1]
        a = jnp.where(a >= 0.0, a, 0.01 * a)
        m = jnp.max(a)
        ex = jnp.exp(a - m)
        z_norm = jnp.sum(ex)
        seg = lax.broadcasted_iota(jnp.int32, (N, G), 1)
        mask = (batch_ref[...] == seg).astype(jnp.float32)
        counts = jnp.sum(mask, axis=0)
        mw = mask * ex
        pooled_sum = lax.dot_general(
            mw, h2, (((0,), (0,)), ((), ())),
            preferred_element_type=jnp.float32)
        denom = z_norm * jnp.maximum(counts, 1.0)
        pooled = pooled_sum / denom[:, None]
        z = jnp.maximum(
            jnp.dot(pooled, wl1_ref[...], preferred_element_type=jnp.float32)
            + bl1_ref[0:1, :], 0.0)
        z = jnp.maximum(
            jnp.dot(z, wl2_ref[...], preferred_element_type=jnp.float32)
            + bl2_ref[0:1, :], 0.0)
        logits = (jnp.dot(z, wc_ref[...], preferred_element_type=jnp.float32)
                  + bc_ref[0:1, :])[:, 0:C]
        lmax = jnp.max(logits, axis=1, keepdims=True)
        le = jnp.exp(logits - lmax)
        o_ref[...] = le / jnp.sum(le, axis=1, keepdims=True)

    full = lambda shape: pl.BlockSpec(shape, lambda i: tuple(0 for _ in shape))
    return pl.pallas_call(
        body,
        grid=(1,),
        in_specs=[
            full((NC, N, F)),
            full((N, F)),
            full((N, 1)),
            full((N, 1)),
            full((8, F)),
            full((F, 8)),
            full((8, F)),
            full((F, F)),
            full((8, F)),
            full((F, F)),
            full((8, F)),
            full((F, F)),
            full((8, F)),
        ],
        out_specs=full((G, C)),
        out_shape=jax.ShapeDtypeStruct((G, C), jnp.float32),
    )(s2, t2d, dinv, batch_col, b2_8, Wa8, ba_8, Wl1, bl1_8, Wl2, bl2_8,
      Wc128, bc_8)


def kernel(x, edge_index, batch, W1, b1, W2, b2, Wa, ba, Wl1, bl1, Wl2, bl2,
           Wc, bc):
    src3 = edge_index[0].astype(jnp.int32).reshape(NW, NCH, K)
    dst3 = edge_index[1].astype(jnp.int32).reshape(NW, NCH, K)
    batch_col = batch.astype(jnp.int32).reshape(N, 1)

    b1_8 = jnp.broadcast_to(b1[None, :], (8, F))
    b2_8 = jnp.broadcast_to(b2[None, :], (8, F))
    bl1_8 = jnp.broadcast_to(bl1[None, :], (8, F))
    bl2_8 = jnp.broadcast_to(bl2[None, :], (8, F))
    ba_8 = jnp.broadcast_to(jnp.reshape(ba, (1, 1)), (8, F))
    bc_8 = jnp.broadcast_to(jnp.pad(bc, (0, F - C))[None, :], (8, F))
    Wa8 = jnp.pad(Wa, ((0, 0), (0, 7)))
    Wc128 = jnp.pad(Wc, ((0, 0), (0, F - C)))

    deg2 = _sc_degree(dst3)
    t1d, dinv = _tc_prescale(deg2, x, W1)
    s1 = _sc_edge_scatter(t1d, src3, dst3)
    t2d = _tc_combine1(s1, t1d, dinv, b1_8, W2)
    s2 = _sc_edge_scatter(t2d, src3, dst3)
    return _tc_head(s2, t2d, dinv, batch_col, b2_8, Wa8, ba_8, Wl1, bl1_8,
                    Wl2, bl2_8, Wc128, bc_8)


# speedup vs baseline: 25.4329x; 1.3011x over previous
import functools

import jax
import jax.numpy as jnp
from jax import lax
from jax.experimental import pallas as pl
from jax.experimental.pallas import tpu as pltpu
from jax.experimental.pallas import tpu_sc as plsc

N = 10000
E = 320000
F = 128
G = 16
C = 2

NC = 2
NS = 16
NW = NC * NS
EPT = E // NW
RPT = N // NS
DW = 16

K = 100
NCH = EPT // K
IBLK = 20
NBLK = NCH // IBLK
NPAIR = IBLK // 2

KD = 80
NCHD = EPT // KD

_vec_mesh = plsc.VectorSubcoreMesh(core_axis_name="c", subcore_axis_name="s")
_sc_params = pltpu.CompilerParams(use_tc_tiling_on_sc=False)


@functools.partial(
    pl.kernel,
    out_type=jax.ShapeDtypeStruct((NC, N, DW), jnp.float32),
    mesh=_vec_mesh,
    scratch_types=[
        pltpu.VMEM((NCHD, KD), jnp.int32),
        pltpu.VMEM((KD, DW), jnp.float32),
        pltpu.VMEM((RPT, DW), jnp.float32),
        pltpu.VMEM_SHARED((N, DW), jnp.float32),
    ],
    compiler_params=_sc_params,
)
def _sc_degree(dst_hbm, out_hbm, dst_v, ones_v, zero_v, acc_deg):
    cid = lax.axis_index("c")
    sid = lax.axis_index("s")
    wid = cid * NS + sid

    pltpu.sync_copy(dst_hbm.at[wid], dst_v)

    @pl.loop(0, KD)
    def _(r):
        ones_v[r, :] = jnp.ones((DW,), jnp.float32)

    @pl.loop(0, RPT)
    def _(r):
        zero_v[r, :] = jnp.zeros((DW,), jnp.float32)

    pltpu.sync_copy(zero_v, acc_deg.at[pl.ds(sid * RPT, RPT)])
    plsc.subcore_barrier()

    @pl.loop(0, NCHD)
    def _(j):
        pltpu.sync_copy(ones_v, acc_deg.at[dst_v.at[j]], add=True)

    plsc.subcore_barrier()
    pltpu.sync_copy(acc_deg.at[pl.ds(sid * RPT, RPT)], zero_v)
    pltpu.sync_copy(zero_v, out_hbm.at[cid, pl.ds(sid * RPT, RPT)])


@functools.partial(
    pl.kernel,
    out_type=jax.ShapeDtypeStruct((NC, N, F), jnp.float32),
    mesh=_vec_mesh,
    scratch_types=[
        pltpu.VMEM((IBLK, K), jnp.int32),
        pltpu.VMEM((IBLK, K), jnp.int32),
        pltpu.VMEM((K, F), jnp.float32),
        pltpu.VMEM((K, F), jnp.float32),
        pltpu.VMEM_SHARED((N, F), jnp.float32),
        pltpu.SemaphoreType.DMA,
        pltpu.SemaphoreType.DMA,
        pltpu.SemaphoreType.DMA,
        pltpu.SemaphoreType.DMA,
    ],
    compiler_params=_sc_params,
)
def _sc_edge_scatter(table_hbm, src_hbm, dst_hbm, out_hbm,
                     src_v, dst_v, rows_a, rows_b, acc,
                     sem_ga, sem_gb, sem_sa, sem_sb):
    cid = lax.axis_index("c")
    sid = lax.axis_index("s")
    wid = cid * NS + sid
    base = sid * RPT

    @pl.loop(0, K)
    def _(r):
        @pl.loop(0, F // 16)
        def _(j):
            rows_a[r, pl.ds(j * 16, 16)] = jnp.zeros((16,), jnp.float32)

    @pl.loop(0, RPT // K)
    def _(b):
        pltpu.sync_copy(rows_a, acc.at[pl.ds(base + b * K, K)])

    pltpu.sync_copy(rows_a.at[pl.ds(0, RPT % K)],
                    acc.at[pl.ds(base + (RPT // K) * K, RPT % K)])

    plsc.subcore_barrier()

    def gather_start(q, buf, sem):
        pltpu.async_copy(table_hbm.at[src_v.at[q]], buf, sem)

    def gather_wait(buf, sem):
        pltpu.make_async_copy(table_hbm.at[pl.ds(0, K)], buf, sem).wait()

    def scatter_start(q, buf, sem):
        pltpu.async_copy(buf, acc.at[dst_v.at[q]], sem, add=True)

    def scatter_wait(buf, sem):
        pltpu.make_async_copy(buf, acc.at[pl.ds(0, K)], sem).wait()

    @pl.loop(0, NBLK)
    def _(blk):
        pltpu.sync_copy(src_hbm.at[wid, pl.ds(blk * IBLK, IBLK)], src_v)
        pltpu.sync_copy(dst_hbm.at[wid, pl.ds(blk * IBLK, IBLK)], dst_v)
        gather_start(0, rows_a, sem_ga)

        @pl.loop(0, NPAIR)
        def _(p):
            gather_wait(rows_a, sem_ga)
            gather_start(2 * p + 1, rows_b, sem_gb)
            scatter_start(2 * p, rows_a, sem_sa)
            gather_wait(rows_b, sem_gb)

            @pl.when(p < NPAIR - 1)
            def _():
                scatter_wait(rows_a, sem_sa)
                gather_start(2 * p + 2, rows_a, sem_ga)

            scatter_start(2 * p + 1, rows_b, sem_sb)
            scatter_wait(rows_b, sem_sb)

            @pl.when(p == NPAIR - 1)
            def _():
                scatter_wait(rows_a, sem_sa)

    plsc.subcore_barrier()

    @pl.loop(0, RPT // K)
    def _(b):
        pltpu.sync_copy(acc.at[pl.ds(base + b * K, K)], rows_a)
        pltpu.sync_copy(rows_a, out_hbm.at[cid, pl.ds(base + b * K, K)])

    pltpu.sync_copy(acc.at[pl.ds(base + (RPT // K) * K, RPT % K)],
                    rows_a.at[pl.ds(0, RPT % K)])
    pltpu.sync_copy(rows_a.at[pl.ds(0, RPT % K)],
                    out_hbm.at[cid, pl.ds(base + (RPT // K) * K, RPT % K)])


def _tc_prescale(deg2, x, W1):
    RB = 1000

    def body(deg_ref, x_ref, w_ref, t1d_ref, dinv_ref):
        deg = deg_ref[0, :, 0:1] + deg_ref[1, :, 0:1] + 1.0
        dinv = lax.rsqrt(jnp.maximum(deg, 1.0))
        t = jnp.dot(x_ref[...], w_ref[...], preferred_element_type=jnp.float32)
        t1d_ref[...] = t * dinv
        dinv_ref[...] = dinv

    return pl.pallas_call(
        body,
        grid=(N // RB,),
        in_specs=[
            pl.BlockSpec((NC, RB, DW), lambda i: (0, i, 0)),
            pl.BlockSpec((RB, F), lambda i: (i, 0)),
            pl.BlockSpec((F, F), lambda i: (0, 0)),
        ],
        out_specs=[
            pl.BlockSpec((RB, F), lambda i: (i, 0)),
            pl.BlockSpec((RB, 1), lambda i: (i, 0)),
        ],
        out_shape=[
            jax.ShapeDtypeStruct((N, F), jnp.float32),
            jax.ShapeDtypeStruct((N, 1), jnp.float32),
        ],
    )(deg2, x, W1)


def _tc_combine1(s1, t1d, dinv, b1_8, W2):
    RB = 1000

    def body(s_ref, t1d_ref, dinv_ref, b1_ref, w2_ref, t2d_ref):
        s = s_ref[0] + s_ref[1] + t1d_ref[...]
        h1 = jnp.maximum(s * dinv_ref[...] + b1_ref[0:1, :], 0.0)
        t2 = jnp.dot(h1, w2_ref[...], preferred_element_type=jnp.float32)
        t2d_ref[...] = t2 * dinv_ref[...]

    return pl.pallas_call(
        body,
        grid=(N // RB,),
        in_specs=[
            pl.BlockSpec((NC, RB, F), lambda i: (0, i, 0)),
            pl.BlockSpec((RB, F), lambda i: (i, 0)),
            pl.BlockSpec((RB, 1), lambda i: (i, 0)),
            pl.BlockSpec((8, F), lambda i: (0, 0)),
            pl.BlockSpec((F, F), lambda i: (0, 0)),
        ],
        out_specs=pl.BlockSpec((RB, F), lambda i: (i, 0)),
        out_shape=jax.ShapeDtypeStruct((N, F), jnp.float32),
    )(s1, t1d, dinv, b1_8, W2)


def _tc_head(s2, t2d, dinv, batch_col, b2_8, Wa8, ba_8, Wl1, bl1_8,
             Wl2, bl2_8, Wc128, bc_8):
    def body(s_ref, t2d_ref, dinv_ref, batch_ref, b2_ref, wa_ref, ba_ref,
             wl1_ref, bl1_ref, wl2_ref, bl2_ref, wc_ref, bc_ref, o_ref):
        h2 = ((s_ref[0] + s_ref[1] + t2d_ref[...]) * dinv_ref[...]
              + b2_ref[0:1, :])
        a8 = jnp.dot(h2, wa_ref[...], preferred_element_type=jnp.float32)
        a = a8[:, 0:1] + ba_ref[0:1, 0:1]
        a = jnp.where(a >= 0.0, a, 0.01 * a)
        m = jnp.max(a)
        ex = jnp.exp(a - m)
        z_norm = jnp.sum(ex)
        seg = lax.broadcasted_iota(jnp.int32, (N, G), 1)
        mask = (batch_ref[...] == seg).astype(jnp.float32)
        counts = jnp.sum(mask, axis=0)
        mw = mask * ex
        pooled_sum = lax.dot_general(
            mw, h2, (((0,), (0,)), ((), ())),
            preferred_element_type=jnp.float32)
        denom = z_norm * jnp.maximum(counts, 1.0)
        pooled = pooled_sum / denom[:, None]
        z = jnp.maximum(
            jnp.dot(pooled, wl1_ref[...], preferred_element_type=jnp.float32)
            + bl1_ref[0:1, :], 0.0)
        z = jnp.maximum(
            jnp.dot(z, wl2_ref[...], preferred_element_type=jnp.float32)
            + bl2_ref[0:1, :], 0.0)
        logits = (jnp.dot(z, wc_ref[...], preferred_element_type=jnp.float32)
                  + bc_ref[0:1, :])[:, 0:C]
        lmax = jnp.max(logits, axis=1, keepdims=True)
        le = jnp.exp(logits - lmax)
        o_ref[...] = le / jnp.sum(le, axis=1, keepdims=True)

    full = lambda shape: pl.BlockSpec(shape, lambda i: tuple(0 for _ in shape))
    return pl.pallas_call(
        body,
        grid=(1,),
        in_specs=[
            full((NC, N, F)),
            full((N, F)),
            full((N, 1)),
            full((N, 1)),
            full((8, F)),
            full((F, 8)),
            full((8, F)),
            full((F, F)),
            full((8, F)),
            full((F, F)),
            full((8, F)),
            full((F, F)),
            full((8, F)),
        ],
        out_specs=full((G, C)),
        out_shape=jax.ShapeDtypeStruct((G, C), jnp.float32),
    )(s2, t2d, dinv, batch_col, b2_8, Wa8, ba_8, Wl1, bl1_8, Wl2, bl2_8,
      Wc128, bc_8)


def kernel(x, edge_index, batch, W1, b1, W2, b2, Wa, ba, Wl1, bl1, Wl2, bl2,
           Wc, bc):
    src3 = edge_index[0].astype(jnp.int32).reshape(NW, NCH, K)
    dst3 = edge_index[1].astype(jnp.int32).reshape(NW, NCH, K)
    dst3d = edge_index[1].astype(jnp.int32).reshape(NW, NCHD, KD)
    batch_col = batch.astype(jnp.int32).reshape(N, 1)

    b1_8 = jnp.broadcast_to(b1[None, :], (8, F))
    b2_8 = jnp.broadcast_to(b2[None, :], (8, F))
    bl1_8 = jnp.broadcast_to(bl1[None, :], (8, F))
    bl2_8 = jnp.broadcast_to(bl2[None, :], (8, F))
    ba_8 = jnp.broadcast_to(jnp.reshape(ba, (1, 1)), (8, F))
    bc_8 = jnp.broadcast_to(jnp.pad(bc, (0, F - C))[None, :], (8, F))
    Wa8 = jnp.pad(Wa, ((0, 0), (0, 7)))
    Wc128 = jnp.pad(Wc, ((0, 0), (0, F - C)))

    deg2 = _sc_degree(dst3d)
    t1d, dinv = _tc_prescale(deg2, x, W1)
    s1 = _sc_edge_scatter(t1d, src3, dst3)
    t2d = _tc_combine1(s1, t1d, dinv, b1_8, W2)
    s2 = _sc_edge_scatter(t2d, src3, dst3)
    return _tc_head(s2, t2d, dinv, batch_col, b2_8, Wa8, ba_8, Wl1, bl1_8,
                    Wl2, bl2_8, Wc128, bc_8)


# speedup vs baseline: 27.1319x; 1.0668x over previous
import functools

import jax
import jax.numpy as jnp
from jax import lax
from jax.experimental import pallas as pl
from jax.experimental.pallas import tpu as pltpu
from jax.experimental.pallas import tpu_sc as plsc

N = 10000
E = 320000
F = 128
G = 16
C = 2

NC = 2
NS = 16
NW = NC * NS
EPT = E // NW
RPT = N // NS
DW = 16

K = 125
NCH = EPT // K
IBLK = 16
NBLK = NCH // IBLK
NPAIR = IBLK // 2

KD = 80
NCHD = EPT // KD

_vec_mesh = plsc.VectorSubcoreMesh(core_axis_name="c", subcore_axis_name="s")
_sc_params = pltpu.CompilerParams(use_tc_tiling_on_sc=False)


@functools.partial(
    pl.kernel,
    out_type=jax.ShapeDtypeStruct((NC, N, DW), jnp.float32),
    mesh=_vec_mesh,
    scratch_types=[
        pltpu.VMEM((NCHD, KD), jnp.int32),
        pltpu.VMEM((KD, DW), jnp.float32),
        pltpu.VMEM((RPT, DW), jnp.float32),
        pltpu.VMEM_SHARED((N, DW), jnp.float32),
    ],
    compiler_params=_sc_params,
)
def _sc_degree(dst_hbm, out_hbm, dst_v, ones_v, zero_v, acc_deg):
    cid = lax.axis_index("c")
    sid = lax.axis_index("s")
    wid = cid * NS + sid

    pltpu.sync_copy(dst_hbm.at[wid], dst_v)

    @pl.loop(0, KD)
    def _(r):
        ones_v[r, :] = jnp.ones((DW,), jnp.float32)

    @pl.loop(0, RPT)
    def _(r):
        zero_v[r, :] = jnp.zeros((DW,), jnp.float32)

    pltpu.sync_copy(zero_v, acc_deg.at[pl.ds(sid * RPT, RPT)])
    plsc.subcore_barrier()

    @pl.loop(0, NCHD)
    def _(j):
        pltpu.sync_copy(ones_v, acc_deg.at[dst_v.at[j]], add=True)

    plsc.subcore_barrier()
    pltpu.sync_copy(acc_deg.at[pl.ds(sid * RPT, RPT)], zero_v)
    pltpu.sync_copy(zero_v, out_hbm.at[cid, pl.ds(sid * RPT, RPT)])


@functools.partial(
    pl.kernel,
    out_type=jax.ShapeDtypeStruct((NC, N, F), jnp.float32),
    mesh=_vec_mesh,
    scratch_types=[
        pltpu.VMEM((IBLK, K), jnp.int32),
        pltpu.VMEM((IBLK, K), jnp.int32),
        pltpu.VMEM((K, F), jnp.float32),
        pltpu.VMEM((K, F), jnp.float32),
        pltpu.VMEM_SHARED((N, F), jnp.float32),
        pltpu.SemaphoreType.DMA,
        pltpu.SemaphoreType.DMA,
        pltpu.SemaphoreType.DMA,
        pltpu.SemaphoreType.DMA,
    ],
    compiler_params=_sc_params,
)
def _sc_edge_scatter(table_hbm, src_hbm, dst_hbm, out_hbm,
                     src_v, dst_v, rows_a, rows_b, acc,
                     sem_ga, sem_gb, sem_sa, sem_sb):
    cid = lax.axis_index("c")
    sid = lax.axis_index("s")
    wid = cid * NS + sid
    base = sid * RPT

    @pl.loop(0, K)
    def _(r):
        @pl.loop(0, F // 16)
        def _(j):
            rows_a[r, pl.ds(j * 16, 16)] = jnp.zeros((16,), jnp.float32)

    @pl.loop(0, RPT // K)
    def _(b):
        pltpu.sync_copy(rows_a, acc.at[pl.ds(base + b * K, K)])

    plsc.subcore_barrier()

    def gather_start(q, buf, sem):
        pltpu.async_copy(table_hbm.at[src_v.at[q]], buf, sem)

    def gather_wait(buf, sem):
        pltpu.make_async_copy(table_hbm.at[pl.ds(0, K)], buf, sem).wait()

    def scatter_start(q, buf, sem):
        pltpu.async_copy(buf, acc.at[dst_v.at[q]], sem, add=True)

    def scatter_wait(buf, sem):
        pltpu.make_async_copy(buf, acc.at[pl.ds(0, K)], sem).wait()

    @pl.loop(0, NBLK)
    def _(blk):
        pltpu.sync_copy(src_hbm.at[wid, pl.ds(blk * IBLK, IBLK)], src_v)
        pltpu.sync_copy(dst_hbm.at[wid, pl.ds(blk * IBLK, IBLK)], dst_v)
        gather_start(0, rows_a, sem_ga)

        @pl.loop(0, NPAIR)
        def _(p):
            gather_wait(rows_a, sem_ga)
            gather_start(2 * p + 1, rows_b, sem_gb)
            scatter_start(2 * p, rows_a, sem_sa)
            gather_wait(rows_b, sem_gb)

            @pl.when(p < NPAIR - 1)
            def _():
                scatter_wait(rows_a, sem_sa)
                gather_start(2 * p + 2, rows_a, sem_ga)

            scatter_start(2 * p + 1, rows_b, sem_sb)
            scatter_wait(rows_b, sem_sb)

            @pl.when(p == NPAIR - 1)
            def _():
                scatter_wait(rows_a, sem_sa)

    plsc.subcore_barrier()

    pltpu.sync_copy(acc.at[pl.ds(base, RPT)],
                    out_hbm.at[cid, pl.ds(base, RPT)])


def _tc_prescale(deg2, x, W1):
    RB = 1000

    def body(deg_ref, x_ref, w_ref, t1d_ref, dinv_ref):
        deg = deg_ref[0, :, 0:1] + deg_ref[1, :, 0:1] + 1.0
        dinv = lax.rsqrt(jnp.maximum(deg, 1.0))
        t = jnp.dot(x_ref[...], w_ref[...], preferred_element_type=jnp.float32)
        t1d_ref[...] = t * dinv
        dinv_ref[...] = dinv

    return pl.pallas_call(
        body,
        grid=(N // RB,),
        in_specs=[
            pl.BlockSpec((NC, RB, DW), lambda i: (0, i, 0)),
            pl.BlockSpec((RB, F), lambda i: (i, 0)),
            pl.BlockSpec((F, F), lambda i: (0, 0)),
        ],
        out_specs=[
            pl.BlockSpec((RB, F), lambda i: (i, 0)),
            pl.BlockSpec((RB, 1), lambda i: (i, 0)),
        ],
        out_shape=[
            jax.ShapeDtypeStruct((N, F), jnp.float32),
            jax.ShapeDtypeStruct((N, 1), jnp.float32),
        ],
    )(deg2, x, W1)


def _tc_combine1(s1, t1d, dinv, b1_8, W2):
    RB = 1000

    def body(s_ref, t1d_ref, dinv_ref, b1_ref, w2_ref, t2d_ref):
        s = s_ref[0] + s_ref[1] + t1d_ref[...]
        h1 = jnp.maximum(s * dinv_ref[...] + b1_ref[0:1, :], 0.0)
        t2 = jnp.dot(h1, w2_ref[...], preferred_element_type=jnp.float32)
        t2d_ref[...] = t2 * dinv_ref[...]

    return pl.pallas_call(
        body,
        grid=(N // RB,),
        in_specs=[
            pl.BlockSpec((NC, RB, F), lambda i: (0, i, 0)),
            pl.BlockSpec((RB, F), lambda i: (i, 0)),
            pl.BlockSpec((RB, 1), lambda i: (i, 0)),
            pl.BlockSpec((8, F), lambda i: (0, 0)),
            pl.BlockSpec((F, F), lambda i: (0, 0)),
        ],
        out_specs=pl.BlockSpec((RB, F), lambda i: (i, 0)),
        out_shape=jax.ShapeDtypeStruct((N, F), jnp.float32),
    )(s1, t1d, dinv, b1_8, W2)


def _tc_head(s2, t2d, dinv, batch_col, b2_8, Wa8, ba_8, Wl1, bl1_8,
             Wl2, bl2_8, Wc128, bc_8):
    def body(s_ref, t2d_ref, dinv_ref, batch_ref, b2_ref, wa_ref, ba_ref,
             wl1_ref, bl1_ref, wl2_ref, bl2_ref, wc_ref, bc_ref, o_ref):
        h2 = ((s_ref[0] + s_ref[1] + t2d_ref[...]) * dinv_ref[...]
              + b2_ref[0:1, :])
        a8 = jnp.dot(h2, wa_ref[...], preferred_element_type=jnp.float32)
        a = a8[:, 0:1] + ba_ref[0:1, 0:1]
        a = jnp.where(a >= 0.0, a, 0.01 * a)
        m = jnp.max(a)
        ex = jnp.exp(a - m)
        z_norm = jnp.sum(ex)
        seg = lax.broadcasted_iota(jnp.int32, (N, G), 1)
        mask = (batch_ref[...] == seg).astype(jnp.float32)
        counts = jnp.sum(mask, axis=0)
        mw = mask * ex
        pooled_sum = lax.dot_general(
            mw, h2, (((0,), (0,)), ((), ())),
            preferred_element_type=jnp.float32)
        denom = z_norm * jnp.maximum(counts, 1.0)
        pooled = pooled_sum / denom[:, None]
        z = jnp.maximum(
            jnp.dot(pooled, wl1_ref[...], preferred_element_type=jnp.float32)
            + bl1_ref[0:1, :], 0.0)
        z = jnp.maximum(
            jnp.dot(z, wl2_ref[...], preferred_element_type=jnp.float32)
            + bl2_ref[0:1, :], 0.0)
        logits = (jnp.dot(z, wc_ref[...], preferred_element_type=jnp.float32)
                  + bc_ref[0:1, :])[:, 0:C]
        lmax = jnp.max(logits, axis=1, keepdims=True)
        le = jnp.exp(logits - lmax)
        o_ref[...] = le / jnp.sum(le, axis=1, keepdims=True)

    full = lambda shape: pl.BlockSpec(shape, lambda i: tuple(0 for _ in shape))
    return pl.pallas_call(
        body,
        grid=(1,),
        in_specs=[
            full((NC, N, F)),
            full((N, F)),
            full((N, 1)),
            full((N, 1)),
            full((8, F)),
            full((F, 8)),
            full((8, F)),
            full((F, F)),
            full((8, F)),
            full((F, F)),
            full((8, F)),
            full((F, F)),
            full((8, F)),
        ],
        out_specs=full((G, C)),
        out_shape=jax.ShapeDtypeStruct((G, C), jnp.float32),
    )(s2, t2d, dinv, batch_col, b2_8, Wa8, ba_8, Wl1, bl1_8, Wl2, bl2_8,
      Wc128, bc_8)


def kernel(x, edge_index, batch, W1, b1, W2, b2, Wa, ba, Wl1, bl1, Wl2, bl2,
           Wc, bc):
    src3 = edge_index[0].astype(jnp.int32).reshape(NW, NCH, K)
    dst3 = edge_index[1].astype(jnp.int32).reshape(NW, NCH, K)
    dst3d = edge_index[1].astype(jnp.int32).reshape(NW, NCHD, KD)
    batch_col = batch.astype(jnp.int32).reshape(N, 1)

    b1_8 = jnp.broadcast_to(b1[None, :], (8, F))
    b2_8 = jnp.broadcast_to(b2[None, :], (8, F))
    bl1_8 = jnp.broadcast_to(bl1[None, :], (8, F))
    bl2_8 = jnp.broadcast_to(bl2[None, :], (8, F))
    ba_8 = jnp.broadcast_to(jnp.reshape(ba, (1, 1)), (8, F))
    bc_8 = jnp.broadcast_to(jnp.pad(bc, (0, F - C))[None, :], (8, F))
    Wa8 = jnp.pad(Wa, ((0, 0), (0, 7)))
    Wc128 = jnp.pad(Wc, ((0, 0), (0, F - C)))

    deg2 = _sc_degree(dst3d)
    t1d, dinv = _tc_prescale(deg2, x, W1)
    s1 = _sc_edge_scatter(t1d, src3, dst3)
    t2d = _tc_combine1(s1, t1d, dinv, b1_8, W2)
    s2 = _sc_edge_scatter(t2d, src3, dst3)
    return _tc_head(s2, t2d, dinv, batch_col, b2_8, Wa8, ba_8, Wl1, bl1_8,
                    Wl2, bl2_8, Wc128, bc_8)


# speedup vs baseline: 28.2457x; 1.0411x over previous
import functools

import jax
import jax.numpy as jnp
from jax import lax
from jax.experimental import pallas as pl
from jax.experimental.pallas import tpu as pltpu
from jax.experimental.pallas import tpu_sc as plsc

N = 10000
E = 320000
F = 128
G = 16
C = 2

NC = 2
NS = 16
NW = NC * NS
EPT = E // NW
RPT = N // NS
DW = 16

K = 125
NCH = EPT // K
IBLK = 16
NBLK = NCH // IBLK
NPAIR = IBLK // 2

KD = 80
NCHD = EPT // KD

_vec_mesh = plsc.VectorSubcoreMesh(core_axis_name="c", subcore_axis_name="s")
_sc_params = pltpu.CompilerParams(use_tc_tiling_on_sc=False)


@functools.partial(
    pl.kernel,
    out_type=jax.ShapeDtypeStruct((NC, N, DW), jnp.float32),
    mesh=_vec_mesh,
    scratch_types=[
        pltpu.VMEM((NCHD, KD), jnp.int32),
        pltpu.VMEM((KD, DW), jnp.float32),
        pltpu.VMEM((RPT, DW), jnp.float32),
        pltpu.VMEM_SHARED((N, DW), jnp.float32),
    ],
    compiler_params=_sc_params,
)
def _sc_degree(dst_hbm, out_hbm, dst_v, ones_v, zero_v, acc_deg):
    cid = lax.axis_index("c")
    sid = lax.axis_index("s")
    wid = cid * NS + sid

    pltpu.sync_copy(dst_hbm.at[wid], dst_v)

    @pl.loop(0, KD)
    def _(r):
        ones_v[r, :] = jnp.ones((DW,), jnp.float32)

    @pl.loop(0, RPT)
    def _(r):
        zero_v[r, :] = jnp.zeros((DW,), jnp.float32)

    pltpu.sync_copy(zero_v, acc_deg.at[pl.ds(sid * RPT, RPT)])
    plsc.subcore_barrier()

    @pl.loop(0, NCHD)
    def _(j):
        pltpu.sync_copy(ones_v, acc_deg.at[dst_v.at[j]], add=True)

    plsc.subcore_barrier()
    pltpu.sync_copy(acc_deg.at[pl.ds(sid * RPT, RPT)], zero_v)
    pltpu.sync_copy(zero_v, out_hbm.at[cid, pl.ds(sid * RPT, RPT)])


@functools.partial(
    pl.kernel,
    out_type=jax.ShapeDtypeStruct((NC, N, F), jnp.bfloat16),
    mesh=_vec_mesh,
    scratch_types=[
        pltpu.VMEM((IBLK, K), jnp.int32),
        pltpu.VMEM((IBLK, K), jnp.int32),
        pltpu.VMEM((K, F), jnp.bfloat16),
        pltpu.VMEM((K, F), jnp.bfloat16),
        pltpu.VMEM_SHARED((N, F), jnp.bfloat16),
        pltpu.SemaphoreType.DMA,
        pltpu.SemaphoreType.DMA,
        pltpu.SemaphoreType.DMA,
        pltpu.SemaphoreType.DMA,
    ],
    compiler_params=_sc_params,
)
def _sc_edge_scatter(table_hbm, src_hbm, dst_hbm, out_hbm,
                     src_v, dst_v, rows_a, rows_b, acc,
                     sem_ga, sem_gb, sem_sa, sem_sb):
    cid = lax.axis_index("c")
    sid = lax.axis_index("s")
    wid = cid * NS + sid
    base = sid * RPT

    @pl.loop(0, K)
    def _(r):
        @pl.loop(0, F // 32)
        def _(j):
            rows_a[r, pl.ds(j * 32, 32)] = jnp.zeros((32,), jnp.bfloat16)

    @pl.loop(0, RPT // K)
    def _(b):
        pltpu.sync_copy(rows_a, acc.at[pl.ds(base + b * K, K)])

    plsc.subcore_barrier()

    def gather_start(q, buf, sem):
        pltpu.async_copy(table_hbm.at[src_v.at[q]], buf, sem)

    def gather_wait(buf, sem):
        pltpu.make_async_copy(table_hbm.at[pl.ds(0, K)], buf, sem).wait()

    def scatter_start(q, buf, sem):
        pltpu.async_copy(buf, acc.at[dst_v.at[q]], sem, add=True)

    def scatter_wait(buf, sem):
        pltpu.make_async_copy(buf, acc.at[pl.ds(0, K)], sem).wait()

    @pl.loop(0, NBLK)
    def _(blk):
        pltpu.sync_copy(src_hbm.at[wid, pl.ds(blk * IBLK, IBLK)], src_v)
        pltpu.sync_copy(dst_hbm.at[wid, pl.ds(blk * IBLK, IBLK)], dst_v)
        gather_start(0, rows_a, sem_ga)

        @pl.loop(0, NPAIR)
        def _(p):
            gather_wait(rows_a, sem_ga)
            gather_start(2 * p + 1, rows_b, sem_gb)
            scatter_start(2 * p, rows_a, sem_sa)
            gather_wait(rows_b, sem_gb)

            @pl.when(p < NPAIR - 1)
            def _():
                scatter_wait(rows_a, sem_sa)
                gather_start(2 * p + 2, rows_a, sem_ga)

            scatter_start(2 * p + 1, rows_b, sem_sb)
            scatter_wait(rows_b, sem_sb)

            @pl.when(p == NPAIR - 1)
            def _():
                scatter_wait(rows_a, sem_sa)

    plsc.subcore_barrier()

    pltpu.sync_copy(acc.at[pl.ds(base, RPT)],
                    out_hbm.at[cid, pl.ds(base, RPT)])


def _tc_prescale(deg2, x, W1):
    RB = 1000

    def body(deg_ref, x_ref, w_ref, t1d_ref, dinv_ref):
        deg = deg_ref[0, :, 0:1] + deg_ref[1, :, 0:1] + 1.0
        dinv = lax.rsqrt(jnp.maximum(deg, 1.0))
        t = jnp.dot(x_ref[...], w_ref[...], preferred_element_type=jnp.float32)
        t1d_ref[...] = (t * dinv).astype(jnp.bfloat16)
        dinv_ref[...] = dinv

    return pl.pallas_call(
        body,
        grid=(N // RB,),
        in_specs=[
            pl.BlockSpec((NC, RB, DW), lambda i: (0, i, 0)),
            pl.BlockSpec((RB, F), lambda i: (i, 0)),
            pl.BlockSpec((F, F), lambda i: (0, 0)),
        ],
        out_specs=[
            pl.BlockSpec((RB, F), lambda i: (i, 0)),
            pl.BlockSpec((RB, 1), lambda i: (i, 0)),
        ],
        out_shape=[
            jax.ShapeDtypeStruct((N, F), jnp.bfloat16),
            jax.ShapeDtypeStruct((N, 1), jnp.float32),
        ],
    )(deg2, x, W1)


def _tc_combine1(s1, t1d, dinv, b1_8, W2):
    RB = 1000

    def body(s_ref, t1d_ref, dinv_ref, b1_ref, w2_ref, t2d_ref):
        s = (s_ref[0].astype(jnp.float32) + s_ref[1].astype(jnp.float32)
             + t1d_ref[...].astype(jnp.float32))
        h1 = jnp.maximum(s * dinv_ref[...] + b1_ref[0:1, :], 0.0)
        t2 = jnp.dot(h1, w2_ref[...], preferred_element_type=jnp.float32)
        t2d_ref[...] = (t2 * dinv_ref[...]).astype(jnp.bfloat16)

    return pl.pallas_call(
        body,
        grid=(N // RB,),
        in_specs=[
            pl.BlockSpec((NC, RB, F), lambda i: (0, i, 0)),
            pl.BlockSpec((RB, F), lambda i: (i, 0)),
            pl.BlockSpec((RB, 1), lambda i: (i, 0)),
            pl.BlockSpec((8, F), lambda i: (0, 0)),
            pl.BlockSpec((F, F), lambda i: (0, 0)),
        ],
        out_specs=pl.BlockSpec((RB, F), lambda i: (i, 0)),
        out_shape=jax.ShapeDtypeStruct((N, F), jnp.bfloat16),
    )(s1, t1d, dinv, b1_8, W2)


def _tc_head(s2, t2d, dinv, batch_col, b2_8, Wa8, ba_8, Wl1, bl1_8,
             Wl2, bl2_8, Wc128, bc_8):
    def body(s_ref, t2d_ref, dinv_ref, batch_ref, b2_ref, wa_ref, ba_ref,
             wl1_ref, bl1_ref, wl2_ref, bl2_ref, wc_ref, bc_ref, o_ref):
        h2 = ((s_ref[0].astype(jnp.float32) + s_ref[1].astype(jnp.float32)
               + t2d_ref[...].astype(jnp.float32)) * dinv_ref[...]
              + b2_ref[0:1, :])
        a8 = jnp.dot(h2, wa_ref[...], preferred_element_type=jnp.float32)
        a = a8[:, 0:1] + ba_ref[0:1, 0:1]
        a = jnp.where(a >= 0.0, a, 0.01 * a)
        m = jnp.max(a)
        ex = jnp.exp(a - m)
        z_norm = jnp.sum(ex)
        seg = lax.broadcasted_iota(jnp.int32, (N, G), 1)
        mask = (batch_ref[...] == seg).astype(jnp.float32)
        counts = jnp.sum(mask, axis=0)
        mw = mask * ex
        pooled_sum = lax.dot_general(
            mw, h2, (((0,), (0,)), ((), ())),
            preferred_element_type=jnp.float32)
        denom = z_norm * jnp.maximum(counts, 1.0)
        pooled = pooled_sum / denom[:, None]
        z = jnp.maximum(
            jnp.dot(pooled, wl1_ref[...], preferred_element_type=jnp.float32)
            + bl1_ref[0:1, :], 0.0)
        z = jnp.maximum(
            jnp.dot(z, wl2_ref[...], preferred_element_type=jnp.float32)
            + bl2_ref[0:1, :], 0.0)
        logits = (jnp.dot(z, wc_ref[...], preferred_element_type=jnp.float32)
                  + bc_ref[0:1, :])[:, 0:C]
        lmax = jnp.max(logits, axis=1, keepdims=True)
        le = jnp.exp(logits - lmax)
        o_ref[...] = le / jnp.sum(le, axis=1, keepdims=True)

    full = lambda shape: pl.BlockSpec(shape, lambda i: tuple(0 for _ in shape))
    return pl.pallas_call(
        body,
        grid=(1,),
        in_specs=[
            full((NC, N, F)),
            full((N, F)),
            full((N, 1)),
            full((N, 1)),
            full((8, F)),
            full((F, 8)),
            full((8, F)),
            full((F, F)),
            full((8, F)),
            full((F, F)),
            full((8, F)),
            full((F, F)),
            full((8, F)),
        ],
        out_specs=full((G, C)),
        out_shape=jax.ShapeDtypeStruct((G, C), jnp.float32),
    )(s2, t2d, dinv, batch_col, b2_8, Wa8, ba_8, Wl1, bl1_8, Wl2, bl2_8,
      Wc128, bc_8)


def kernel(x, edge_index, batch, W1, b1, W2, b2, Wa, ba, Wl1, bl1, Wl2, bl2,
           Wc, bc):
    src3 = edge_index[0].astype(jnp.int32).reshape(NW, NCH, K)
    dst3 = edge_index[1].astype(jnp.int32).reshape(NW, NCH, K)
    dst3d = edge_index[1].astype(jnp.int32).reshape(NW, NCHD, KD)
    batch_col = batch.astype(jnp.int32).reshape(N, 1)

    b1_8 = jnp.broadcast_to(b1[None, :], (8, F))
    b2_8 = jnp.broadcast_to(b2[None, :], (8, F))
    bl1_8 = jnp.broadcast_to(bl1[None, :], (8, F))
    bl2_8 = jnp.broadcast_to(bl2[None, :], (8, F))
    ba_8 = jnp.broadcast_to(jnp.reshape(ba, (1, 1)), (8, F))
    bc_8 = jnp.broadcast_to(jnp.pad(bc, (0, F - C))[None, :], (8, F))
    Wa8 = jnp.pad(Wa, ((0, 0), (0, 7)))
    Wc128 = jnp.pad(Wc, ((0, 0), (0, F - C)))

    deg2 = _sc_degree(dst3d)
    t1d, dinv = _tc_prescale(deg2, x, W1)
    s1 = _sc_edge_scatter(t1d, src3, dst3)
    t2d = _tc_combine1(s1, t1d, dinv, b1_8, W2)
    s2 = _sc_edge_scatter(t2d, src3, dst3)
    return _tc_head(s2, t2d, dinv, batch_col, b2_8, Wa8, ba_8, Wl1, bl1_8,
                    Wl2, bl2_8, Wc128, bc_8)


# speedup vs baseline: 29.4509x; 1.0427x over previous
import functools

import jax
import jax.numpy as jnp
from jax import lax
from jax.experimental import pallas as pl
from jax.experimental.pallas import tpu as pltpu
from jax.experimental.pallas import tpu_sc as plsc

N = 10000
E = 320000
F = 128
G = 16
C = 2

NC = 2
NS = 16
NW = NC * NS
EPT = E // NW
RPT = N // NS
DW = 16

K = 125
NCH = EPT // K
NPAIR = NCH // 2

KD = 100
NCHD = EPT // KD
NPAIRD = NCHD // 2

_vec_mesh = plsc.VectorSubcoreMesh(core_axis_name="c", subcore_axis_name="s")
_sc_params = pltpu.CompilerParams(use_tc_tiling_on_sc=False)


@functools.partial(
    pl.kernel,
    out_type=jax.ShapeDtypeStruct((NC, N, DW), jnp.float32),
    mesh=_vec_mesh,
    scratch_types=[
        pltpu.VMEM((NCHD, KD), jnp.int32),
        pltpu.VMEM((KD, DW), jnp.float32),
        pltpu.VMEM((RPT, DW), jnp.float32),
        pltpu.VMEM_SHARED((N, DW), jnp.float32),
        pltpu.SemaphoreType.DMA,
        pltpu.SemaphoreType.DMA,
    ],
    compiler_params=_sc_params,
)
def _sc_degree(dst_hbm, out_hbm, dst_v, ones_v, zero_v, acc_deg,
               sem_a, sem_b):
    cid = lax.axis_index("c")
    sid = lax.axis_index("s")
    wid = cid * NS + sid

    pltpu.sync_copy(dst_hbm.at[wid], dst_v)

    @pl.loop(0, KD)
    def _(r):
        ones_v[r, :] = jnp.ones((DW,), jnp.float32)

    @pl.loop(0, RPT)
    def _(r):
        zero_v[r, :] = jnp.zeros((DW,), jnp.float32)

    pltpu.sync_copy(zero_v, acc_deg.at[pl.ds(sid * RPT, RPT)])
    plsc.subcore_barrier()

    def dscat_start(q, sem):
        pltpu.async_copy(ones_v, acc_deg.at[dst_v.at[q]], sem, add=True)

    def dscat_wait(sem):
        pltpu.make_async_copy(ones_v, acc_deg.at[pl.ds(0, KD)], sem).wait()

    dscat_start(0, sem_a)

    @pl.loop(0, NPAIRD)
    def _(p):
        dscat_start(2 * p + 1, sem_b)
        dscat_wait(sem_a)

        @pl.when(p < NPAIRD - 1)
        def _():
            dscat_start(2 * p + 2, sem_a)

        dscat_wait(sem_b)

    plsc.subcore_barrier()
    pltpu.sync_copy(acc_deg.at[pl.ds(sid * RPT, RPT)], zero_v)
    pltpu.sync_copy(zero_v, out_hbm.at[cid, pl.ds(sid * RPT, RPT)])


@functools.partial(
    pl.kernel,
    out_type=jax.ShapeDtypeStruct((NC, N, F), jnp.bfloat16),
    mesh=_vec_mesh,
    scratch_types=[
        pltpu.VMEM((NCH, K), jnp.int32),
        pltpu.VMEM((NCH, K), jnp.int32),
        pltpu.VMEM((K, F), jnp.bfloat16),
        pltpu.VMEM((K, F), jnp.bfloat16),
        pltpu.VMEM_SHARED((N, F), jnp.bfloat16),
        pltpu.SemaphoreType.DMA,
        pltpu.SemaphoreType.DMA,
        pltpu.SemaphoreType.DMA,
        pltpu.SemaphoreType.DMA,
    ],
    compiler_params=_sc_params,
)
def _sc_edge_scatter(table_hbm, src_hbm, dst_hbm, out_hbm,
                     src_v, dst_v, rows_a, rows_b, acc,
                     sem_ga, sem_gb, sem_sa, sem_sb):
    cid = lax.axis_index("c")
    sid = lax.axis_index("s")
    wid = cid * NS + sid
    base = sid * RPT

    pltpu.sync_copy(src_hbm.at[wid], src_v)
    pltpu.sync_copy(dst_hbm.at[wid], dst_v)

    @pl.loop(0, K)
    def _(r):
        @pl.loop(0, F // 32)
        def _(j):
            rows_a[r, pl.ds(j * 32, 32)] = jnp.zeros((32,), jnp.bfloat16)

    @pl.loop(0, RPT // K)
    def _(b):
        pltpu.sync_copy(rows_a, acc.at[pl.ds(base + b * K, K)])

    plsc.subcore_barrier()

    def gather_start(q, buf, sem):
        pltpu.async_copy(table_hbm.at[src_v.at[q]], buf, sem)

    def gather_wait(buf, sem):
        pltpu.make_async_copy(table_hbm.at[pl.ds(0, K)], buf, sem).wait()

    def scatter_start(q, buf, sem):
        pltpu.async_copy(buf, acc.at[dst_v.at[q]], sem, add=True)

    def scatter_wait(buf, sem):
        pltpu.make_async_copy(buf, acc.at[pl.ds(0, K)], sem).wait()

    gather_start(0, rows_a, sem_ga)

    @pl.loop(0, NPAIR)
    def _(p):
        gather_wait(rows_a, sem_ga)
        gather_start(2 * p + 1, rows_b, sem_gb)
        scatter_start(2 * p, rows_a, sem_sa)
        gather_wait(rows_b, sem_gb)

        @pl.when(p < NPAIR - 1)
        def _():
            scatter_wait(rows_a, sem_sa)
            gather_start(2 * p + 2, rows_a, sem_ga)

        scatter_start(2 * p + 1, rows_b, sem_sb)
        scatter_wait(rows_b, sem_sb)

        @pl.when(p == NPAIR - 1)
        def _():
            scatter_wait(rows_a, sem_sa)

    plsc.subcore_barrier()

    pltpu.sync_copy(acc.at[pl.ds(base, RPT)],
                    out_hbm.at[cid, pl.ds(base, RPT)])


def _tc_prescale(deg2, x, W1):
    RB = 1000

    def body(deg_ref, x_ref, w_ref, t1d_ref, dinv_ref):
        deg = deg_ref[0, :, 0:1] + deg_ref[1, :, 0:1] + 1.0
        dinv = lax.rsqrt(jnp.maximum(deg, 1.0))
        t = jnp.dot(x_ref[...], w_ref[...], preferred_element_type=jnp.float32)
        t1d_ref[...] = (t * dinv).astype(jnp.bfloat16)
        dinv_ref[...] = dinv

    return pl.pallas_call(
        body,
        grid=(N // RB,),
        in_specs=[
            pl.BlockSpec((NC, RB, DW), lambda i: (0, i, 0)),
            pl.BlockSpec((RB, F), lambda i: (i, 0)),
            pl.BlockSpec((F, F), lambda i: (0, 0)),
        ],
        out_specs=[
            pl.BlockSpec((RB, F), lambda i: (i, 0)),
            pl.BlockSpec((RB, 1), lambda i: (i, 0)),
        ],
        out_shape=[
            jax.ShapeDtypeStruct((N, F), jnp.bfloat16),
            jax.ShapeDtypeStruct((N, 1), jnp.float32),
        ],
    )(deg2, x, W1)


def _tc_combine1(s1, t1d, dinv, b1_8, W2):
    RB = 1000

    def body(s_ref, t1d_ref, dinv_ref, b1_ref, w2_ref, t2d_ref):
        s = (s_ref[0].astype(jnp.float32) + s_ref[1].astype(jnp.float32)
             + t1d_ref[...].astype(jnp.float32))
        h1 = jnp.maximum(s * dinv_ref[...] + b1_ref[0:1, :], 0.0)
        t2 = jnp.dot(h1, w2_ref[...], preferred_element_type=jnp.float32)
        t2d_ref[...] = (t2 * dinv_ref[...]).astype(jnp.bfloat16)

    return pl.pallas_call(
        body,
        grid=(N // RB,),
        in_specs=[
            pl.BlockSpec((NC, RB, F), lambda i: (0, i, 0)),
            pl.BlockSpec((RB, F), lambda i: (i, 0)),
            pl.BlockSpec((RB, 1), lambda i: (i, 0)),
            pl.BlockSpec((8, F), lambda i: (0, 0)),
            pl.BlockSpec((F, F), lambda i: (0, 0)),
        ],
        out_specs=pl.BlockSpec((RB, F), lambda i: (i, 0)),
        out_shape=jax.ShapeDtypeStruct((N, F), jnp.bfloat16),
    )(s1, t1d, dinv, b1_8, W2)


def _tc_head(s2, t2d, dinv, batch_col, b2_8, Wa8, ba_8, Wl1, bl1_8,
             Wl2, bl2_8, Wc128, bc_8):
    def body(s_ref, t2d_ref, dinv_ref, batch_ref, b2_ref, wa_ref, ba_ref,
             wl1_ref, bl1_ref, wl2_ref, bl2_ref, wc_ref, bc_ref, o_ref):
        h2 = ((s_ref[0].astype(jnp.float32) + s_ref[1].astype(jnp.float32)
               + t2d_ref[...].astype(jnp.float32)) * dinv_ref[...]
              + b2_ref[0:1, :])
        a8 = jnp.dot(h2, wa_ref[...], preferred_element_type=jnp.float32)
        a = a8[:, 0:1] + ba_ref[0:1, 0:1]
        a = jnp.where(a >= 0.0, a, 0.01 * a)
        m = jnp.max(a)
        ex = jnp.exp(a - m)
        z_norm = jnp.sum(ex)
        seg = lax.broadcasted_iota(jnp.int32, (N, G), 1)
        mask = (batch_ref[...] == seg).astype(jnp.float32)
        counts = jnp.sum(mask, axis=0)
        mw = mask * ex
        pooled_sum = lax.dot_general(
            mw, h2, (((0,), (0,)), ((), ())),
            preferred_element_type=jnp.float32)
        denom = z_norm * jnp.maximum(counts, 1.0)
        pooled = pooled_sum / denom[:, None]
        z = jnp.maximum(
            jnp.dot(pooled, wl1_ref[...], preferred_element_type=jnp.float32)
            + bl1_ref[0:1, :], 0.0)
        z = jnp.maximum(
            jnp.dot(z, wl2_ref[...], preferred_element_type=jnp.float32)
            + bl2_ref[0:1, :], 0.0)
        logits = (jnp.dot(z, wc_ref[...], preferred_element_type=jnp.float32)
                  + bc_ref[0:1, :])[:, 0:C]
        lmax = jnp.max(logits, axis=1, keepdims=True)
        le = jnp.exp(logits - lmax)
        o_ref[...] = le / jnp.sum(le, axis=1, keepdims=True)

    full = lambda shape: pl.BlockSpec(shape, lambda i: tuple(0 for _ in shape))
    return pl.pallas_call(
        body,
        grid=(1,),
        in_specs=[
            full((NC, N, F)),
            full((N, F)),
            full((N, 1)),
            full((N, 1)),
            full((8, F)),
            full((F, 8)),
            full((8, F)),
            full((F, F)),
            full((8, F)),
            full((F, F)),
            full((8, F)),
            full((F, F)),
            full((8, F)),
        ],
        out_specs=full((G, C)),
        out_shape=jax.ShapeDtypeStruct((G, C), jnp.float32),
    )(s2, t2d, dinv, batch_col, b2_8, Wa8, ba_8, Wl1, bl1_8, Wl2, bl2_8,
      Wc128, bc_8)


def kernel(x, edge_index, batch, W1, b1, W2, b2, Wa, ba, Wl1, bl1, Wl2, bl2,
           Wc, bc):
    src3 = edge_index[0].astype(jnp.int32).reshape(NW, NCH, K)
    dst3 = edge_index[1].astype(jnp.int32).reshape(NW, NCH, K)
    dst3d = edge_index[1].astype(jnp.int32).reshape(NW, NCHD, KD)
    batch_col = batch.astype(jnp.int32).reshape(N, 1)

    b1_8 = jnp.broadcast_to(b1[None, :], (8, F))
    b2_8 = jnp.broadcast_to(b2[None, :], (8, F))
    bl1_8 = jnp.broadcast_to(bl1[None, :], (8, F))
    bl2_8 = jnp.broadcast_to(bl2[None, :], (8, F))
    ba_8 = jnp.broadcast_to(jnp.reshape(ba, (1, 1)), (8, F))
    bc_8 = jnp.broadcast_to(jnp.pad(bc, (0, F - C))[None, :], (8, F))
    Wa8 = jnp.pad(Wa, ((0, 0), (0, 7)))
    Wc128 = jnp.pad(Wc, ((0, 0), (0, F - C)))

    deg2 = _sc_degree(dst3d)
    t1d, dinv = _tc_prescale(deg2, x, W1)
    s1 = _sc_edge_scatter(t1d, src3, dst3)
    t2d = _tc_combine1(s1, t1d, dinv, b1_8, W2)
    s2 = _sc_edge_scatter(t2d, src3, dst3)
    return _tc_head(s2, t2d, dinv, batch_col, b2_8, Wa8, ba_8, Wl1, bl1_8,
                    Wl2, bl2_8, Wc128, bc_8)
